# Initial kernel scaffold; baseline (speedup 1.0000x reference)
import jax
import jax.numpy as jnp
from jax.experimental import pallas as pl


def kernel(x, x_subset, knn_index, W, b):
    raise NotImplementedError("write your pallas kernel here")


# speedup vs baseline: 5.6684x; 5.6684x over previous
import functools

import jax
import jax.numpy as jnp
from jax import lax
from jax.experimental import pallas as pl
from jax.experimental.pallas import tpu as pltpu
from jax.experimental.pallas import tpu_sc as plsc

P = 10000
K = 32
D = 128
OUT = 128

NC = 2
NS = 16
NW = NC * NS
P_PAD = 10240
PTS_W = P_PAD // NW
GRP = 4
NG = PTS_W // GRP
IDX_ROWS_W = PTS_W * K // 128


def _mm_body(x_ref, xs_ref, w_ref, b_ref, z_ref, y_ref):
    w1 = w_ref[:D, :]
    wd = w_ref[D:, :] - w1
    z_ref[...] = jnp.dot(x_ref[...], w1, preferred_element_type=jnp.float32)
    y_ref[...] = (
        jnp.dot(xs_ref[...], wd, preferred_element_type=jnp.float32)
        + b_ref[...]
    )


def _fin_body(m_ref, y_ref, o_ref):
    o_ref[...] = jnp.maximum(m_ref[...] + y_ref[...], 0.0).T


def _sc_body(z_hbm, idx_hbm, m_hbm, idx_v, buf0, buf1, out_v, sem0, sem1):
    c = lax.axis_index("c")
    s = lax.axis_index("s")
    wid = s * NC + c
    base_idx_row = wid * IDX_ROWS_W
    base_pt = wid * PTS_W

    pltpu.sync_copy(idx_hbm.at[pl.ds(base_idx_row, IDX_ROWS_W)], idx_v)

    bufs = (buf0, buf1)
    sems = (sem0, sem1)

    def _gather(g, buf, sem):
        return pltpu.make_async_copy(z_hbm.at[idx_v.at[g]], buf, sem)

    _gather(0, buf0, sem0).start()
    _gather(1, buf1, sem1).start()

    def _compute_group(g, buf):
        for i in range(GRP):
            row0 = i * K
            accs = [buf[row0, pl.ds(d * 16, 16)] for d in range(8)]
            for k in range(1, K):
                for d in range(8):
                    accs[d] = jnp.maximum(
                        accs[d], buf[row0 + k, pl.ds(d * 16, 16)]
                    )
            for d in range(8):
                out_v[g * GRP + i, pl.ds(d * 16, 16)] = accs[d]

    def _step(it, carry):
        for lane in range(2):
            g = it * 2 + lane
            buf, sem = bufs[lane], sems[lane]
            _gather(g, buf, sem).wait()
            _compute_group(g, buf)

            @pl.when(g + 2 < NG)
            def _():
                _gather(g + 2, buf, sem).start()

        return carry

    lax.fori_loop(0, NG // 2, _step, 0)
    pltpu.sync_copy(out_v, m_hbm.at[pl.ds(base_pt, PTS_W)])


def kernel(x, x_subset, knn_index, W, b):
    xf = x[0]
    xsf = x_subset[0]
    idx_flat = knn_index[0].reshape(P * K)
    idx_pad = jnp.pad(idx_flat, (0, P_PAD * K - P * K))
    idx2d = idx_pad.reshape(P_PAD * K // 128, 128)
    b2d = b.reshape(1, OUT)

    blk = 1000
    grid = P // blk
    z, y = pl.pallas_call(
        _mm_body,
        grid=(grid,),
        in_specs=[
            pl.BlockSpec((blk, D), lambda i: (i, 0)),
            pl.BlockSpec((blk, D), lambda i: (i, 0)),
            pl.BlockSpec((2 * D, OUT), lambda i: (0, 0)),
            pl.BlockSpec((1, OUT), lambda i: (0, 0)),
        ],
        out_specs=[
            pl.BlockSpec((blk, OUT), lambda i: (i, 0)),
            pl.BlockSpec((blk, OUT), lambda i: (i, 0)),
        ],
        out_shape=[
            jax.ShapeDtypeStruct((P, OUT), jnp.float32),
            jax.ShapeDtypeStruct((P, OUT), jnp.float32),
        ],
    )(xf, xsf, W, b2d)

    mesh = plsc.VectorSubcoreMesh(
        core_axis_name="c", subcore_axis_name="s", num_cores=NC, num_subcores=NS
    )
    m = pl.kernel(
        _sc_body,
        out_type=jax.ShapeDtypeStruct((P_PAD, OUT), jnp.float32),
        mesh=mesh,
        scratch_types=[
            pltpu.VMEM((IDX_ROWS_W, 128), jnp.int32),
            pltpu.VMEM((GRP * K, OUT), jnp.float32),
            pltpu.VMEM((GRP * K, OUT), jnp.float32),
            pltpu.VMEM((PTS_W, OUT), jnp.float32),
            pltpu.SemaphoreType.DMA,
            pltpu.SemaphoreType.DMA,
        ],
    )(z, idx2d)

    out = pl.pallas_call(
        _fin_body,
        out_shape=jax.ShapeDtypeStruct((OUT, P), jnp.float32),
    )(m[:P], y)

    return out[None]
